# baseline (speedup 1.0000x reference)
import functools

import jax
import jax.numpy as jnp
from jax import lax
from jax.experimental import pallas as pl
from jax.experimental.pallas import tpu as pltpu
from jax.experimental.pallas import tpu_sc as plsc

N = 10000
E = 320000
D_IN = 128
D_H = 256
D = 128
K = 10

NW = 32
EPW = E // NW
CH = 40
NCH = -(-EPW // CH)
EPW_PAD = NCH * CH
NPAD = EPW_PAD - EPW

NP = 10112
BR = 632
SLAB = NP // 16
DUMP_BASE = N
DEGR = 10240

_MESH = plsc.VectorSubcoreMesh(core_axis_name="c", subcore_axis_name="s")


def _load_mask_edges(row_hbm, col_hbm, wid, ridx, cidx):
    e0 = wid * EPW
    pltpu.sync_copy(row_hbm.at[pl.ds(e0, EPW)], ridx.at[pl.ds(0, EPW)])
    pltpu.sync_copy(col_hbm.at[pl.ds(e0, EPW)], cidx.at[pl.ds(0, EPW)])
    lane = lax.iota(jnp.int32, 16)
    for t in range(NPAD // 16):
        ridx[pl.ds(EPW + 16 * t, 16)] = lane + (16 * t)
        cidx[pl.ds(EPW + 16 * t, 16)] = lane + (DUMP_BASE + (16 * t) % 64)

    def mask_group(g, carry):
        r = ridx[pl.ds(g * 16, 16)]
        c = cidx[pl.ds(g * 16, 16)]
        dump = DUMP_BASE + jnp.bitwise_and(c, 63)
        cidx[pl.ds(g * 16, 16)] = jnp.where(r == c, dump, c)
        return carry

    lax.fori_loop(0, EPW_PAD // 16, mask_group, 0)


@functools.partial(
    pl.kernel,
    mesh=_MESH,
    out_type=(
        jax.ShapeDtypeStruct((2 * DEGR,), jnp.float32),
        jax.ShapeDtypeStruct((NW * EPW_PAD,), jnp.int32),
        jax.ShapeDtypeStruct((NW * EPW_PAD,), jnp.int32),
    ),
    scratch_types=[
        pltpu.VMEM((EPW_PAD,), jnp.int32),
        pltpu.VMEM((EPW_PAD,), jnp.int32),
        pltpu.VMEM((CH,), jnp.float32),
        pltpu.VMEM_SHARED((DEGR,), jnp.float32),
        pltpu.SemaphoreType.DMA,
    ],
)
def _sc_deg(row_hbm, col_hbm, z1_hbm, deg_hbm, mrow_hbm, mcol_hbm,
            ridx, cidx, ones, dacc, sem):
    c = lax.axis_index("c")
    s = lax.axis_index("s")
    wid = s * 2 + c
    for t in range(CH // 16):
        ones[pl.ds(16 * t, 16)] = jnp.full((16,), 1.0, jnp.float32)
    pltpu.sync_copy(z1_hbm.at[pl.ds(s * 640, 640)], dacc.at[pl.ds(s * 640, 640)])
    _load_mask_edges(row_hbm, col_hbm, wid, ridx, cidx)
    pltpu.sync_copy(ridx, mrow_hbm.at[pl.ds(wid * EPW_PAD, EPW_PAD)])
    pltpu.sync_copy(cidx, mcol_hbm.at[pl.ds(wid * EPW_PAD, EPW_PAD)])
    plsc.subcore_barrier()

    def edge_chunk(j, carry):
        pltpu.async_copy(ones, dacc.at[cidx.at[pl.ds(j * CH, CH)]], sem,
                         add=True)
        return carry

    lax.fori_loop(0, NCH, edge_chunk, 0)
    pltpu.make_async_copy(row_hbm.at[pl.ds(0, EPW_PAD)],
                          ridx.at[pl.ds(0, EPW_PAD)], sem).wait()
    plsc.subcore_barrier()
    pltpu.sync_copy(dacc.at[pl.ds(s * 640, 640)], deg_hbm.at[pl.ds(c * DEGR + s * 640, 640)])


@functools.partial(
    pl.kernel,
    mesh=_MESH,
    out_type=jax.ShapeDtypeStruct((2, NP, D), jnp.float32),
    scratch_types=[
        pltpu.VMEM((EPW_PAD,), jnp.int32),
        pltpu.VMEM((EPW_PAD,), jnp.int32),
        pltpu.VMEM((4 * CH, D), jnp.float32),
        pltpu.VMEM_SHARED((NP, D), jnp.float32),
        pltpu.SemaphoreType.DMA,
        pltpu.SemaphoreType.DMA,
        pltpu.SemaphoreType.DMA,
        pltpu.SemaphoreType.DMA,
    ],
)
def _sc_step(mrow_hbm, mcol_hbm, curp_hbm, z_hbm, out_hbm,
             ridx, cidx, gbuf, acc, m0, m1, m2, m3):
    c = lax.axis_index("c")
    s = lax.axis_index("s")
    wid = s * 2 + c
    c1 = pltpu.async_copy(z_hbm.at[pl.ds(s * SLAB, SLAB)],
                          acc.at[pl.ds(s * SLAB, SLAB)], m0)
    c2 = pltpu.async_copy(mrow_hbm.at[pl.ds(wid * EPW_PAD, EPW_PAD)], ridx,
                          m1)
    c3 = pltpu.async_copy(mcol_hbm.at[pl.ds(wid * EPW_PAD, EPW_PAD)], cidx,
                          m2)
    c1.wait()
    c2.wait()
    c3.wait()
    plsc.subcore_barrier()

    sems = (m0, m1, m2, m3)

    def fire_gather(j, slot, sem):
        pltpu.async_copy(
            curp_hbm.at[ridx.at[pl.ds(j * CH, CH)]],
            gbuf.at[pl.ds(slot * CH, CH)], sem)

    def wait_chunk(sem):
        pltpu.make_async_copy(
            z_hbm.at[pl.ds(0, CH)], gbuf.at[pl.ds(0, CH)], sem).wait()

    def fire_scatter(j, slot, sem):
        pltpu.async_copy(gbuf.at[pl.ds(slot * CH, CH)],
                         acc.at[cidx.at[pl.ds(j * CH, CH)]], sem, add=True)

    fire_gather(0, 0, m0)
    fire_gather(1, 1, m1)
    wait_chunk(m0)
    fire_scatter(0, 0, m0)
    fire_gather(2, 2, m2)
    wait_chunk(m1)
    fire_scatter(1, 1, m1)
    fire_gather(3, 3, m3)

    def group(g, carry):
        for p in range(4):
            slot = (2 + p) % 4
            sem = sems[slot]
            qslot = (slot + 2) % 4
            qsem = sems[qslot]
            j = g * 4 + 2 + p
            wait_chunk(sem)
            fire_scatter(j, slot, sem)
            wait_chunk(qsem)
            fire_gather(j + 2, qslot, qsem)
        return carry

    lax.fori_loop(0, (NCH - 4) // 4, group, 0)
    for j in (NCH - 2, NCH - 1):
        slot = j % 4
        wait_chunk(sems[slot])
        fire_scatter(j, slot, sems[slot])
    for p in range(4):
        wait_chunk(sems[p])
    plsc.subcore_barrier()
    pltpu.sync_copy(acc.at[pl.ds(s * SLAB, SLAB)], out_hbm.at[c, pl.ds(s * SLAB, SLAB)])


def _tc_mlp(x_pad, W1, b1r, W2, b2r):
    def body(x_ref, w1_ref, b1_ref, w2_ref, b2_ref, h_ref):
        h = jnp.maximum(x_ref[...] @ w1_ref[...] + b1_ref[...], 0.0)
        h_ref[...] = h @ w2_ref[...] + b2_ref[...]

    return pl.pallas_call(
        body,
        grid=(NP // BR,),
        in_specs=[
            pl.BlockSpec((BR, D_IN), lambda i: (i, 0)),
            pl.BlockSpec((D_IN, D_H), lambda i: (0, 0)),
            pl.BlockSpec((1, D_H), lambda i: (0, 0)),
            pl.BlockSpec((D_H, D), lambda i: (0, 0)),
            pl.BlockSpec((1, D), lambda i: (0, 0)),
        ],
        out_specs=[pl.BlockSpec((BR, D), lambda i: (i, 0))],
        out_shape=[jax.ShapeDtypeStruct((NP, D), jnp.float32)],
    )(x_pad, W1, b1r, W2, b2r)[0]


def _tc_scale(h, degdump):
    def body(h_ref, deg_ref, curp0_ref, dis_ref):
        deg = deg_ref[0, :, 0] + deg_ref[1, :, 0] + 1.0
        dis = jnp.where(deg > 0, lax.rsqrt(deg), 0.0)[:, None]
        curp0_ref[...] = h_ref[...] * dis
        dis_ref[...] = dis

    return pl.pallas_call(
        body,
        grid=(NP // BR,),
        in_specs=[
            pl.BlockSpec((BR, D), lambda i: (i, 0)),
            pl.BlockSpec((2, BR, 1), lambda i: (0, i, 0)),
        ],
        out_specs=[
            pl.BlockSpec((BR, D), lambda i: (i, 0)),
            pl.BlockSpec((BR, 1), lambda i: (i, 0)),
        ],
        out_shape=[
            jax.ShapeDtypeStruct((NP, D), jnp.float32),
            jax.ShapeDtypeStruct((NP, 1), jnp.float32),
        ],
    )(h, degdump)


def _tc_step(accdump, curp, dis):
    def body(acc_ref, curp_ref, dis_ref, pred_ref, curpn_ref):
        a = acc_ref[0] + acc_ref[1] + curp_ref[...]
        p = a * dis_ref[...]
        pred_ref[...] = p
        curpn_ref[...] = p * dis_ref[...]

    return pl.pallas_call(
        body,
        grid=(NP // BR,),
        in_specs=[
            pl.BlockSpec((2, BR, D), lambda i: (0, i, 0)),
            pl.BlockSpec((BR, D), lambda i: (i, 0)),
            pl.BlockSpec((BR, 1), lambda i: (i, 0)),
        ],
        out_specs=[
            pl.BlockSpec((BR, D), lambda i: (i, 0)),
            pl.BlockSpec((BR, D), lambda i: (i, 0)),
        ],
        out_shape=[
            jax.ShapeDtypeStruct((NP, D), jnp.float32),
            jax.ShapeDtypeStruct((NP, D), jnp.float32),
        ],
    )(accdump, curp, dis)


def _tc_final(preds, proj_w, proj_br):
    def body(*refs):
        p_refs = refs[: K + 1]
        pw = refs[K + 1][...]
        pb = refs[K + 2][...]
        logp_ref = refs[K + 3]
        out_ref = refs[K + 4]
        out = jnp.zeros((BR, D), jnp.float32)
        for pr in p_refs:
            p = pr[...]
            score = jax.nn.sigmoid(p @ pw + pb)
            out = out + score * p
        out_ref[...] = out
        m = jnp.max(out, axis=1, keepdims=True)
        lse = jnp.log(jnp.sum(jnp.exp(out - m), axis=1, keepdims=True)) + m
        logp_ref[...] = out - lse

    return pl.pallas_call(
        body,
        grid=(NP // BR,),
        in_specs=[pl.BlockSpec((BR, D), lambda i: (i, 0)) for _ in range(K + 1)]
        + [
            pl.BlockSpec((D, 1), lambda i: (0, 0)),
            pl.BlockSpec((1, 1), lambda i: (0, 0)),
        ],
        out_specs=[
            pl.BlockSpec((BR, D), lambda i: (i, 0)),
            pl.BlockSpec((BR, D), lambda i: (i, 0)),
        ],
        out_shape=[
            jax.ShapeDtypeStruct((NP, D), jnp.float32),
            jax.ShapeDtypeStruct((NP, D), jnp.float32),
        ],
    )(*preds, proj_w, proj_br)


def kernel(x, edge_index, W1, b1, W2, b2, proj_w, proj_b):
    x_pad = jnp.pad(x, ((0, NP - N), (0, 0)))
    b1r = b1.reshape(1, D_H)
    b2r = b2.reshape(1, D)
    pbr = proj_b.reshape(1, 1)
    erow = edge_index[0]
    ecol = edge_index[1]
    z1 = jnp.zeros((DEGR,), jnp.float32)
    z = jnp.zeros((NP, D), jnp.float32)
    degdump, mrow, mcol = _sc_deg(erow, ecol, z1)
    pred0 = _tc_mlp(x_pad, W1, b1r, W2, b2r)
    curp, dis = _tc_scale(pred0, degdump.reshape(2, DEGR, 1))
    preds = [pred0]
    for _ in range(K):
        accdump = _sc_step(mrow, mcol, curp, z)
        pred, curp = _tc_step(accdump, curp, dis)
        preds.append(pred)
    logp, out = _tc_final(preds, proj_w, pbr)
    return logp[:N], out[:N]


# speedup vs baseline: 1.3289x; 1.3289x over previous
import functools

import jax
import jax.numpy as jnp
from jax import lax
from jax.experimental import pallas as pl
from jax.experimental.pallas import tpu as pltpu
from jax.experimental.pallas import tpu_sc as plsc

N = 10000
E = 320000
D_IN = 128
D_H = 256
D = 128
K = 10

NW = 32
EPW = E // NW
CH = 48
NCH = -(-EPW // CH)
EPW_PAD = NCH * CH
NPAD = EPW_PAD - EPW

NP = 10112
BR = 632
SLAB = NP // 16
DUMP_BASE = N
DEGR = 10240

_MESH = plsc.VectorSubcoreMesh(core_axis_name="c", subcore_axis_name="s")


def _load_mask_edges(row_hbm, col_hbm, wid, ridx, cidx):
    e0 = wid * EPW
    pltpu.sync_copy(row_hbm.at[pl.ds(e0, EPW)], ridx.at[pl.ds(0, EPW)])
    pltpu.sync_copy(col_hbm.at[pl.ds(e0, EPW)], cidx.at[pl.ds(0, EPW)])
    lane = lax.iota(jnp.int32, 16)
    for t in range(NPAD // 16):
        ridx[pl.ds(EPW + 16 * t, 16)] = lane + (16 * t)
        cidx[pl.ds(EPW + 16 * t, 16)] = lane + (DUMP_BASE + (16 * t) % 64)

    def mask_group(g, carry):
        r = ridx[pl.ds(g * 16, 16)]
        c = cidx[pl.ds(g * 16, 16)]
        dump = DUMP_BASE + jnp.bitwise_and(c, 63)
        cidx[pl.ds(g * 16, 16)] = jnp.where(r == c, dump, c)
        return carry

    lax.fori_loop(0, EPW_PAD // 16, mask_group, 0)


@functools.partial(
    pl.kernel,
    mesh=_MESH,
    out_type=(
        jax.ShapeDtypeStruct((2 * DEGR,), jnp.float32),
        jax.ShapeDtypeStruct((NW * EPW_PAD,), jnp.int32),
        jax.ShapeDtypeStruct((NW * EPW_PAD,), jnp.int32),
    ),
    scratch_types=[
        pltpu.VMEM((EPW_PAD,), jnp.int32),
        pltpu.VMEM((EPW_PAD,), jnp.int32),
        pltpu.VMEM((CH,), jnp.float32),
        pltpu.VMEM_SHARED((DEGR,), jnp.float32),
        pltpu.SemaphoreType.DMA,
    ],
)
def _sc_deg(row_hbm, col_hbm, z1_hbm, deg_hbm, mrow_hbm, mcol_hbm,
            ridx, cidx, ones, dacc, sem):
    c = lax.axis_index("c")
    s = lax.axis_index("s")
    wid = s * 2 + c
    for t in range(CH // 16):
        ones[pl.ds(16 * t, 16)] = jnp.full((16,), 1.0, jnp.float32)
    pltpu.sync_copy(z1_hbm.at[pl.ds(s * 640, 640)], dacc.at[pl.ds(s * 640, 640)])
    _load_mask_edges(row_hbm, col_hbm, wid, ridx, cidx)
    pltpu.sync_copy(ridx, mrow_hbm.at[pl.ds(wid * EPW_PAD, EPW_PAD)])
    pltpu.sync_copy(cidx, mcol_hbm.at[pl.ds(wid * EPW_PAD, EPW_PAD)])
    plsc.subcore_barrier()

    def edge_chunk(j, carry):
        pltpu.async_copy(ones, dacc.at[cidx.at[pl.ds(j * CH, CH)]], sem,
                         add=True)
        return carry

    lax.fori_loop(0, NCH, edge_chunk, 0)
    pltpu.make_async_copy(row_hbm.at[pl.ds(0, EPW_PAD)],
                          ridx.at[pl.ds(0, EPW_PAD)], sem).wait()
    plsc.subcore_barrier()
    pltpu.sync_copy(dacc.at[pl.ds(s * 640, 640)], deg_hbm.at[pl.ds(c * DEGR + s * 640, 640)])


@functools.partial(
    pl.kernel,
    mesh=_MESH,
    out_type=jax.ShapeDtypeStruct((2, NP, D), jnp.float32),
    scratch_types=[
        pltpu.VMEM((EPW_PAD,), jnp.int32),
        pltpu.VMEM((EPW_PAD,), jnp.int32),
        pltpu.VMEM((4 * CH, D), jnp.float32),
        pltpu.VMEM_SHARED((NP, D), jnp.float32),
        pltpu.SemaphoreType.DMA,
        pltpu.SemaphoreType.DMA,
    ],
)
def _sc_step(mrow_hbm, mcol_hbm, curp_hbm, z_hbm, out_hbm,
             ridx, cidx, gbuf, acc, gsem, ssem):
    c = lax.axis_index("c")
    s = lax.axis_index("s")
    wid = s * 2 + c
    c1 = pltpu.async_copy(z_hbm.at[pl.ds(s * SLAB, SLAB)],
                          acc.at[pl.ds(s * SLAB, SLAB)], gsem)
    c2 = pltpu.async_copy(mrow_hbm.at[pl.ds(wid * EPW_PAD, EPW_PAD)], ridx,
                          ssem)
    c3 = pltpu.async_copy(mcol_hbm.at[pl.ds(wid * EPW_PAD, EPW_PAD)], cidx,
                          gsem)
    c1.wait()
    c2.wait()
    c3.wait()
    plsc.subcore_barrier()

    def fire_gather(j, slot):
        pltpu.async_copy(
            curp_hbm.at[ridx.at[pl.ds(j * CH, CH)]],
            gbuf.at[pl.ds(slot * CH, CH)], gsem)

    def wait_chunk(sem):
        pltpu.make_async_copy(
            z_hbm.at[pl.ds(0, CH)], gbuf.at[pl.ds(0, CH)], sem).wait()

    def fire_scatter(j, slot):
        pltpu.async_copy(gbuf.at[pl.ds(slot * CH, CH)],
                         acc.at[cidx.at[pl.ds(j * CH, CH)]], ssem, add=True)

    fire_gather(0, 0)
    fire_gather(1, 1)
    fire_gather(2, 2)
    wait_chunk(gsem)
    fire_gather(3, 3)
    fire_scatter(0, 0)

    def body(j, carry):
        slot = jnp.bitwise_and(j, 3)
        slot3 = jnp.bitwise_and(j + 3, 3)
        wait_chunk(gsem)
        wait_chunk(ssem)
        fire_gather(j + 3, slot3)
        fire_scatter(j, slot)
        return carry

    lax.fori_loop(1, NCH - 3, body, 0)
    for j in (NCH - 3, NCH - 2, NCH - 1):
        wait_chunk(gsem)
        wait_chunk(ssem)
        fire_scatter(j, j & 3)
    wait_chunk(ssem)
    plsc.subcore_barrier()
    pltpu.sync_copy(acc.at[pl.ds(s * SLAB, SLAB)], out_hbm.at[c, pl.ds(s * SLAB, SLAB)])


def _tc_mlp(x_pad, W1, b1r, W2, b2r):
    def body(x_ref, w1_ref, b1_ref, w2_ref, b2_ref, h_ref):
        h = jnp.maximum(x_ref[...] @ w1_ref[...] + b1_ref[...], 0.0)
        h_ref[...] = h @ w2_ref[...] + b2_ref[...]

    return pl.pallas_call(
        body,
        grid=(NP // BR,),
        in_specs=[
            pl.BlockSpec((BR, D_IN), lambda i: (i, 0)),
            pl.BlockSpec((D_IN, D_H), lambda i: (0, 0)),
            pl.BlockSpec((1, D_H), lambda i: (0, 0)),
            pl.BlockSpec((D_H, D), lambda i: (0, 0)),
            pl.BlockSpec((1, D), lambda i: (0, 0)),
        ],
        out_specs=[pl.BlockSpec((BR, D), lambda i: (i, 0))],
        out_shape=[jax.ShapeDtypeStruct((NP, D), jnp.float32)],
    )(x_pad, W1, b1r, W2, b2r)[0]


def _tc_scale(h, degdump):
    def body(h_ref, deg_ref, curp0_ref, dis_ref):
        deg = deg_ref[0, :, 0] + deg_ref[1, :, 0] + 1.0
        dis = jnp.where(deg > 0, lax.rsqrt(deg), 0.0)[:, None]
        curp0_ref[...] = h_ref[...] * dis
        dis_ref[...] = dis

    return pl.pallas_call(
        body,
        grid=(NP // BR,),
        in_specs=[
            pl.BlockSpec((BR, D), lambda i: (i, 0)),
            pl.BlockSpec((2, BR, 1), lambda i: (0, i, 0)),
        ],
        out_specs=[
            pl.BlockSpec((BR, D), lambda i: (i, 0)),
            pl.BlockSpec((BR, 1), lambda i: (i, 0)),
        ],
        out_shape=[
            jax.ShapeDtypeStruct((NP, D), jnp.float32),
            jax.ShapeDtypeStruct((NP, 1), jnp.float32),
        ],
    )(h, degdump)


def _tc_step(accdump, curp, dis):
    def body(acc_ref, curp_ref, dis_ref, pred_ref, curpn_ref):
        a = acc_ref[0] + acc_ref[1] + curp_ref[...]
        p = a * dis_ref[...]
        pred_ref[...] = p
        curpn_ref[...] = p * dis_ref[...]

    return pl.pallas_call(
        body,
        grid=(NP // BR,),
        in_specs=[
            pl.BlockSpec((2, BR, D), lambda i: (0, i, 0)),
            pl.BlockSpec((BR, D), lambda i: (i, 0)),
            pl.BlockSpec((BR, 1), lambda i: (i, 0)),
        ],
        out_specs=[
            pl.BlockSpec((BR, D), lambda i: (i, 0)),
            pl.BlockSpec((BR, D), lambda i: (i, 0)),
        ],
        out_shape=[
            jax.ShapeDtypeStruct((NP, D), jnp.float32),
            jax.ShapeDtypeStruct((NP, D), jnp.float32),
        ],
    )(accdump, curp, dis)


def _tc_final(preds, proj_w, proj_br):
    def body(*refs):
        p_refs = refs[: K + 1]
        pw = refs[K + 1][...]
        pb = refs[K + 2][...]
        logp_ref = refs[K + 3]
        out_ref = refs[K + 4]
        out = jnp.zeros((BR, D), jnp.float32)
        for pr in p_refs:
            p = pr[...]
            score = jax.nn.sigmoid(p @ pw + pb)
            out = out + score * p
        out_ref[...] = out
        m = jnp.max(out, axis=1, keepdims=True)
        lse = jnp.log(jnp.sum(jnp.exp(out - m), axis=1, keepdims=True)) + m
        logp_ref[...] = out - lse

    return pl.pallas_call(
        body,
        grid=(NP // BR,),
        in_specs=[pl.BlockSpec((BR, D), lambda i: (i, 0)) for _ in range(K + 1)]
        + [
            pl.BlockSpec((D, 1), lambda i: (0, 0)),
            pl.BlockSpec((1, 1), lambda i: (0, 0)),
        ],
        out_specs=[
            pl.BlockSpec((BR, D), lambda i: (i, 0)),
            pl.BlockSpec((BR, D), lambda i: (i, 0)),
        ],
        out_shape=[
            jax.ShapeDtypeStruct((NP, D), jnp.float32),
            jax.ShapeDtypeStruct((NP, D), jnp.float32),
        ],
    )(*preds, proj_w, proj_br)


def kernel(x, edge_index, W1, b1, W2, b2, proj_w, proj_b):
    x_pad = jnp.pad(x, ((0, NP - N), (0, 0)))
    b1r = b1.reshape(1, D_H)
    b2r = b2.reshape(1, D)
    pbr = proj_b.reshape(1, 1)
    erow = edge_index[0]
    ecol = edge_index[1]
    z1 = jnp.zeros((DEGR,), jnp.float32)
    z = jnp.zeros((NP, D), jnp.float32)
    degdump, mrow, mcol = _sc_deg(erow, ecol, z1)
    pred0 = _tc_mlp(x_pad, W1, b1r, W2, b2r)
    curp, dis = _tc_scale(pred0, degdump.reshape(2, DEGR, 1))
    preds = [pred0]
    for _ in range(K):
        accdump = _sc_step(mrow, mcol, curp, z)
        pred, curp = _tc_step(accdump, curp, dis)
        preds.append(pred)
    logp, out = _tc_final(preds, proj_w, pbr)
    return logp[:N], out[:N]
